# baseline (speedup 1.0000x reference)
import functools

import jax
import jax.numpy as jnp
from jax import lax
from jax.experimental import pallas as pl
from jax.experimental.pallas import tpu as pltpu
from jax.experimental.pallas import tpu_sc as plsc

N = 10000
E = 320000
IN_DIM = 128
HIDDEN = 256
OUT_DIM = 128
HALF = IN_DIM // 2

NC = 2
NS = 16
NW = NC * NS
LANES = 128

NBD = 80
E_PAD_D = NW * NBD * LANES

NB = 160
E_PAD_A = NS * NB * LANES
GRP = 4

N_PAD = 10240
ROWS_PER_TILE = N_PAD // NS
WB_CHUNK = 128
BLK = 1024

_mesh = plsc.VectorSubcoreMesh(
    core_axis_name="c", subcore_axis_name="s", num_cores=NC, num_subcores=NS
)


@functools.partial(
    pl.kernel,
    out_type=jax.ShapeDtypeStruct((NC, N_PAD), jnp.float32),
    mesh=_mesh,
    scratch_types=[
        pltpu.VMEM((NBD, LANES), jnp.int32),
        pltpu.VMEM((LANES,), jnp.float32),
        pltpu.VMEM((ROWS_PER_TILE,), jnp.float32),
        pltpu.VMEM_SHARED((N_PAD,), jnp.float32),
    ],
)
def _deg_kernel(dst_hbm, out_hbm, dst_v, ones_v, cbuf, cnt_acc):
    c = lax.axis_index("c")
    s = lax.axis_index("s")
    g = c * NS + s

    pltpu.sync_copy(dst_hbm.at[g], dst_v)

    def fill_ones(i, carry):
        ones_v[pl.ds(i * 16, 16)] = jnp.ones((16,), jnp.float32)
        return carry

    lax.fori_loop(0, LANES // 16, fill_ones, 0)

    def zero_cbuf(i, carry):
        cbuf[pl.ds(i * 16, 16)] = jnp.zeros((16,), jnp.float32)
        return carry

    lax.fori_loop(0, ROWS_PER_TILE // 16, zero_cbuf, 0)

    pltpu.sync_copy(cbuf, cnt_acc.at[pl.ds(s * ROWS_PER_TILE, ROWS_PER_TILE)])
    plsc.subcore_barrier()

    def count_batch(j, carry):
        pltpu.sync_copy(ones_v, cnt_acc.at[dst_v.at[j]], add=True)
        return carry

    lax.fori_loop(0, NBD, count_batch, 0)
    plsc.subcore_barrier()

    pltpu.sync_copy(cnt_acc.at[pl.ds(s * ROWS_PER_TILE, ROWS_PER_TILE)], cbuf)
    pltpu.sync_copy(cbuf, out_hbm.at[c, pl.ds(s * ROWS_PER_TILE, ROWS_PER_TILE)])


@functools.partial(
    pl.kernel,
    out_type=jax.ShapeDtypeStruct((NC, N_PAD, HALF), jnp.float32),
    mesh=_mesh,
    scratch_types=[
        pltpu.VMEM((NB // 2, LANES), jnp.int32),
        pltpu.VMEM((NB // 2, LANES), jnp.int32),
        pltpu.VMEM((2 * GRP, LANES, HALF), jnp.float32),
        pltpu.VMEM_SHARED((N_PAD, HALF), jnp.float32),
        pltpu.SemaphoreType.DMA,
        pltpu.SemaphoreType.DMA,
        pltpu.SemaphoreType.DMA,
        pltpu.SemaphoreType.DMA,
    ],
    compiler_params=pltpu.CompilerParams(use_tc_tiling_on_sc=False),
)
def _agg_kernel(u_hbm, src_hbm, dst_hbm, out_hbm, src_v, dst_v, ring, acc, ga, sa, gb, sb):
    c = lax.axis_index("c")
    s = lax.axis_index("s")

    zbuf = ring.at[0]

    def zero_buf(i, carry):
        ring[0, i // (HALF // 16), pl.ds((i % (HALF // 16)) * 16, 16)] = jnp.zeros(
            (16,), jnp.float32
        )
        return carry

    lax.fori_loop(0, LANES * HALF // 16, zero_buf, 0)
    base = s * ROWS_PER_TILE

    def zero_acc(k, carry):
        pltpu.sync_copy(zbuf, acc.at[pl.ds(base + k * WB_CHUNK, WB_CHUNK)])
        return carry

    lax.fori_loop(0, ROWS_PER_TILE // WB_CHUNK, zero_acc, 0)
    plsc.subcore_barrier()

    uc = u_hbm.at[c]
    nh = NB // 2

    def gath(j, k, sem):
        return pltpu.async_copy(uc.at[src_v.at[j]], ring.at[k], sem)

    def gath_wait(j, k, sem):
        pltpu.make_async_copy(uc.at[src_v.at[j]], ring.at[k], sem).wait()

    def scat(j, k, sem):
        return pltpu.async_copy(ring.at[k], acc.at[dst_v.at[j]], sem, add=True)

    def scat_wait(j, k, sem):
        pltpu.make_async_copy(ring.at[k], acc.at[dst_v.at[j]], sem).wait()

    for h in range(2):
        pltpu.sync_copy(src_hbm.at[s, pl.ds(h * nh, nh)], src_v)
        pltpu.sync_copy(dst_hbm.at[s, pl.ds(h * nh, nh)], dst_v)

        for k in range(GRP):
            gath(k, k, ga)

        def body(ss, carry):
            b0 = 8 * ss

            for k in range(GRP):
                gath(b0 + GRP + k, GRP + k, gb)
            for k in range(GRP):
                gath_wait(b0 + k, k, ga)
            for k in range(GRP):
                scat(b0 + k, k, sa)
            for k in range(GRP):
                gath_wait(b0 + GRP + k, GRP + k, gb)
            for k in range(GRP):
                scat(b0 + GRP + k, GRP + k, sb)
            for k in range(GRP):
                scat_wait(b0 + k, k, sa)

            @pl.when(b0 + 8 < nh)
            def _():
                for k in range(GRP):
                    gath(b0 + 8 + k, k, ga)

            for k in range(GRP):
                scat_wait(b0 + GRP + k, GRP + k, sb)
            return carry

        lax.fori_loop(0, nh // 8, body, 0)

    plsc.subcore_barrier()

    def writeback(k, carry):
        r0 = base + k * WB_CHUNK
        pltpu.sync_copy(acc.at[pl.ds(r0, WB_CHUNK)], zbuf)
        pltpu.sync_copy(zbuf, out_hbm.at[c, pl.ds(r0, WB_CHUNK)])
        return carry

    lax.fori_loop(0, ROWS_PER_TILE // WB_CHUNK, writeback, 0)


def _tc0_body(cnt_ref, x_ref, dinv_ref, u_ref):
    cnt = cnt_ref[...]
    deg = cnt[:, 0:1] + cnt[:, 1:2] + 1.0
    dinv = lax.rsqrt(deg)
    dinv_ref[...] = dinv
    u = x_ref[...] * dinv
    u_ref[0] = u[:, :HALF]
    u_ref[1] = u[:, HALF:]


def _tc1_body(s1_ref, u1_ref, dinv_ref, w1_ref, b1_ref, w2_ref, u2_ref):
    dinv = dinv_ref[...]
    su = jnp.concatenate([s1_ref[0] + u1_ref[0], s1_ref[1] + u1_ref[1]], axis=-1)
    p = su * dinv
    h = jnp.dot(p, w1_ref[...], preferred_element_type=jnp.float32) + b1_ref[...]
    h = jnp.maximum(h, 0.0)
    q = jnp.dot(h, w2_ref[...], preferred_element_type=jnp.float32)
    u2 = q * dinv
    u2_ref[0] = u2[:, :HALF]
    u2_ref[1] = u2[:, HALF:]


def _tc2_body(s2_ref, u2_ref, dinv_ref, b2_ref, wd_ref, bd_ref, emb_ref, recon_ref):
    su = jnp.concatenate([s2_ref[0] + u2_ref[0], s2_ref[1] + u2_ref[1]], axis=-1)
    emb = su * dinv_ref[...] + b2_ref[...]
    emb_ref[...] = emb
    recon_ref[...] = (
        jnp.dot(emb, wd_ref[...], preferred_element_type=jnp.float32) + bd_ref[...]
    )


def _row_spec(cols):
    return pl.BlockSpec((BLK, cols), lambda i: (i, 0))


def _split_spec():
    return pl.BlockSpec((NC, BLK, HALF), lambda i: (0, i, 0))


def _full_spec(shape):
    nd = len(shape)
    return pl.BlockSpec(shape, lambda i: (0,) * nd)


_split_shape = jax.ShapeDtypeStruct((NC, N_PAD, HALF), jnp.float32)

_tc0 = pl.pallas_call(
    _tc0_body,
    grid=(N_PAD // BLK,),
    in_specs=[_row_spec(NC), _row_spec(IN_DIM)],
    out_specs=[_row_spec(1), _split_spec()],
    out_shape=[jax.ShapeDtypeStruct((N_PAD, 1), jnp.float32), _split_shape],
)

_tc1 = pl.pallas_call(
    _tc1_body,
    grid=(N_PAD // BLK,),
    in_specs=[
        _split_spec(),
        _split_spec(),
        _row_spec(1),
        _full_spec((IN_DIM, HIDDEN)),
        _full_spec((1, HIDDEN)),
        _full_spec((HIDDEN, OUT_DIM)),
    ],
    out_specs=[_split_spec()],
    out_shape=[_split_shape],
)

_tc2 = pl.pallas_call(
    _tc2_body,
    grid=(N_PAD // BLK,),
    in_specs=[
        _split_spec(),
        _split_spec(),
        _row_spec(1),
        _full_spec((1, OUT_DIM)),
        _full_spec((OUT_DIM, IN_DIM)),
        _full_spec((1, IN_DIM)),
    ],
    out_specs=[_row_spec(OUT_DIM), _row_spec(IN_DIM)],
    out_shape=[
        jax.ShapeDtypeStruct((N_PAD, OUT_DIM), jnp.float32),
        jax.ShapeDtypeStruct((N_PAD, IN_DIM), jnp.float32),
    ],
)


def kernel(x, edge_index, W1, b1, W2, b2, Wd, bd):
    src = edge_index[0]
    dst = edge_index[1]
    padd = E_PAD_D - E
    dst3d = jnp.concatenate([dst, jnp.full((padd,), N, jnp.int32)]).reshape(
        NW, NBD, LANES
    )
    pada = E_PAD_A - E
    src3 = jnp.concatenate([src, jnp.full((pada,), N + 1, jnp.int32)]).reshape(
        NS, NB, LANES
    )
    dst3 = jnp.concatenate([dst, jnp.full((pada,), N, jnp.int32)]).reshape(
        NS, NB, LANES
    )
    x_pad = jnp.concatenate([x, jnp.zeros((N_PAD - N, IN_DIM), x.dtype)])

    cnt = _deg_kernel(dst3d)
    dinv, u1 = _tc0(cnt.T, x_pad)
    s1 = _agg_kernel(u1, src3, dst3)
    (u2,) = _tc1(s1, u1, dinv, W1, b1.reshape(1, HIDDEN), W2)
    s2 = _agg_kernel(u2, src3, dst3)
    emb_pad, recon_pad = _tc2(
        s2, u2, dinv, b2.reshape(1, OUT_DIM), Wd, bd.reshape(1, IN_DIM)
    )
    return emb_pad[:N], recon_pad[:N]


# speedup vs baseline: 1.0317x; 1.0317x over previous
import functools

import jax
import jax.numpy as jnp
from jax import lax
from jax.experimental import pallas as pl
from jax.experimental.pallas import tpu as pltpu
from jax.experimental.pallas import tpu_sc as plsc

N = 10000
E = 320000
IN_DIM = 128
HIDDEN = 256
OUT_DIM = 128
HALF = IN_DIM // 2

NC = 2
NS = 16
NW = NC * NS
LANES = 128

NBD = 80
E_PAD_D = NW * NBD * LANES

NB = 160
E_PAD_A = NS * NB * LANES
GRP = 4

N_PAD = 10240
ROWS_PER_TILE = N_PAD // NS
WB_CHUNK = 128
BLK = 1024

_mesh = plsc.VectorSubcoreMesh(
    core_axis_name="c", subcore_axis_name="s", num_cores=NC, num_subcores=NS
)


@functools.partial(
    pl.kernel,
    out_type=jax.ShapeDtypeStruct((NC, N_PAD), jnp.float32),
    mesh=_mesh,
    scratch_types=[
        pltpu.VMEM((NBD, LANES), jnp.int32),
        pltpu.VMEM((LANES,), jnp.float32),
        pltpu.VMEM((ROWS_PER_TILE,), jnp.float32),
        pltpu.VMEM_SHARED((N_PAD,), jnp.float32),
    ],
)
def _deg_kernel(dst_hbm, out_hbm, dst_v, ones_v, cbuf, cnt_acc):
    c = lax.axis_index("c")
    s = lax.axis_index("s")
    g = c * NS + s

    pltpu.sync_copy(dst_hbm.at[g], dst_v)

    def fill_ones(i, carry):
        ones_v[pl.ds(i * 16, 16)] = jnp.ones((16,), jnp.float32)
        return carry

    lax.fori_loop(0, LANES // 16, fill_ones, 0)

    def zero_cbuf(i, carry):
        cbuf[pl.ds(i * 16, 16)] = jnp.zeros((16,), jnp.float32)
        return carry

    lax.fori_loop(0, ROWS_PER_TILE // 16, zero_cbuf, 0)

    pltpu.sync_copy(cbuf, cnt_acc.at[pl.ds(s * ROWS_PER_TILE, ROWS_PER_TILE)])
    plsc.subcore_barrier()

    def count_batch(j, carry):
        pltpu.sync_copy(ones_v, cnt_acc.at[dst_v.at[j]], add=True)
        return carry

    lax.fori_loop(0, NBD, count_batch, 0)
    plsc.subcore_barrier()

    pltpu.sync_copy(cnt_acc.at[pl.ds(s * ROWS_PER_TILE, ROWS_PER_TILE)], cbuf)
    pltpu.sync_copy(cbuf, out_hbm.at[c, pl.ds(s * ROWS_PER_TILE, ROWS_PER_TILE)])


@functools.partial(
    pl.kernel,
    out_type=jax.ShapeDtypeStruct((NC, N_PAD, HALF), jnp.float32),
    mesh=_mesh,
    scratch_types=[
        pltpu.VMEM((NB // 2, LANES), jnp.int32),
        pltpu.VMEM((NB // 2, LANES), jnp.int32),
        pltpu.VMEM((2 * GRP, LANES, HALF), jnp.float32),
        pltpu.VMEM_SHARED((N_PAD, HALF), jnp.float32),
        pltpu.SemaphoreType.DMA,
        pltpu.SemaphoreType.DMA,
        pltpu.SemaphoreType.DMA,
        pltpu.SemaphoreType.DMA,
    ],
    compiler_params=pltpu.CompilerParams(use_tc_tiling_on_sc=False),
)
def _agg_kernel(u_hbm, src_hbm, dst_hbm, out_hbm, src_v, dst_v, ring, acc, ga, sa, gb, sb):
    c = lax.axis_index("c")
    s = lax.axis_index("s")

    zbuf = ring.at[0]

    def zero_buf(i, carry):
        ring[0, i // (HALF // 16), pl.ds((i % (HALF // 16)) * 16, 16)] = jnp.zeros(
            (16,), jnp.float32
        )
        return carry

    lax.fori_loop(0, LANES * HALF // 16, zero_buf, 0)
    base = s * ROWS_PER_TILE

    def zero_acc(k, carry):
        pltpu.sync_copy(zbuf, acc.at[pl.ds(base + k * WB_CHUNK, WB_CHUNK)])
        return carry

    lax.fori_loop(0, ROWS_PER_TILE // WB_CHUNK, zero_acc, 0)
    plsc.subcore_barrier()

    uc = u_hbm.at[c]
    nh = NB // 2

    def gath(j, k, sem):
        return pltpu.async_copy(uc.at[src_v.at[j]], ring.at[k], sem)

    def gath_wait(j, k, sem):
        pltpu.make_async_copy(uc.at[src_v.at[j]], ring.at[k], sem).wait()

    def scat(j, k, sem):
        return pltpu.async_copy(ring.at[k], acc.at[dst_v.at[j]], sem, add=True)

    def scat_wait(j, k, sem):
        pltpu.make_async_copy(ring.at[k], acc.at[dst_v.at[j]], sem).wait()

    for h in range(2):
        pltpu.sync_copy(src_hbm.at[s, pl.ds(h * nh, nh)], src_v)
        pltpu.sync_copy(dst_hbm.at[s, pl.ds(h * nh, nh)], dst_v)

        for k in range(GRP):
            gath(k, k, ga)

        def body(ss, carry):
            b0 = 8 * ss

            for k in range(GRP):
                gath(b0 + GRP + k, GRP + k, gb)
            for k in range(GRP):
                gath_wait(b0 + k, k, ga)
            for k in range(GRP):
                pltpu.sync_copy(ring.at[k], acc.at[dst_v.at[b0 + k]], add=True)

            @pl.when(b0 + 8 < nh)
            def _():
                for k in range(GRP):
                    gath(b0 + 8 + k, k, ga)

            for k in range(GRP):
                gath_wait(b0 + GRP + k, GRP + k, gb)
            for k in range(GRP):
                pltpu.sync_copy(
                    ring.at[GRP + k], acc.at[dst_v.at[b0 + GRP + k]], add=True
                )
            return carry

        lax.fori_loop(0, nh // 8, body, 0)

    plsc.subcore_barrier()

    def writeback(k, carry):
        r0 = base + k * WB_CHUNK
        pltpu.sync_copy(acc.at[pl.ds(r0, WB_CHUNK)], zbuf)
        pltpu.sync_copy(zbuf, out_hbm.at[c, pl.ds(r0, WB_CHUNK)])
        return carry

    lax.fori_loop(0, ROWS_PER_TILE // WB_CHUNK, writeback, 0)


def _tc0_body(cnt_ref, x_ref, dinv_ref, u_ref):
    cnt = cnt_ref[...]
    deg = cnt[:, 0:1] + cnt[:, 1:2] + 1.0
    dinv = lax.rsqrt(deg)
    dinv_ref[...] = dinv
    u = x_ref[...] * dinv
    u_ref[0] = u[:, :HALF]
    u_ref[1] = u[:, HALF:]


def _tc1_body(s1_ref, u1_ref, dinv_ref, w1_ref, b1_ref, w2_ref, u2_ref):
    dinv = dinv_ref[...]
    su = jnp.concatenate([s1_ref[0] + u1_ref[0], s1_ref[1] + u1_ref[1]], axis=-1)
    p = su * dinv
    h = jnp.dot(p, w1_ref[...], preferred_element_type=jnp.float32) + b1_ref[...]
    h = jnp.maximum(h, 0.0)
    q = jnp.dot(h, w2_ref[...], preferred_element_type=jnp.float32)
    u2 = q * dinv
    u2_ref[0] = u2[:, :HALF]
    u2_ref[1] = u2[:, HALF:]


def _tc2_body(s2_ref, u2_ref, dinv_ref, b2_ref, wd_ref, bd_ref, emb_ref, recon_ref):
    su = jnp.concatenate([s2_ref[0] + u2_ref[0], s2_ref[1] + u2_ref[1]], axis=-1)
    emb = su * dinv_ref[...] + b2_ref[...]
    emb_ref[...] = emb
    recon_ref[...] = (
        jnp.dot(emb, wd_ref[...], preferred_element_type=jnp.float32) + bd_ref[...]
    )


def _row_spec(cols):
    return pl.BlockSpec((BLK, cols), lambda i: (i, 0))


def _split_spec():
    return pl.BlockSpec((NC, BLK, HALF), lambda i: (0, i, 0))


def _full_spec(shape):
    nd = len(shape)
    return pl.BlockSpec(shape, lambda i: (0,) * nd)


_split_shape = jax.ShapeDtypeStruct((NC, N_PAD, HALF), jnp.float32)

_tc0 = pl.pallas_call(
    _tc0_body,
    grid=(N_PAD // BLK,),
    in_specs=[_row_spec(NC), _row_spec(IN_DIM)],
    out_specs=[_row_spec(1), _split_spec()],
    out_shape=[jax.ShapeDtypeStruct((N_PAD, 1), jnp.float32), _split_shape],
)

_tc1 = pl.pallas_call(
    _tc1_body,
    grid=(N_PAD // BLK,),
    in_specs=[
        _split_spec(),
        _split_spec(),
        _row_spec(1),
        _full_spec((IN_DIM, HIDDEN)),
        _full_spec((1, HIDDEN)),
        _full_spec((HIDDEN, OUT_DIM)),
    ],
    out_specs=[_split_spec()],
    out_shape=[_split_shape],
)

_tc2 = pl.pallas_call(
    _tc2_body,
    grid=(N_PAD // BLK,),
    in_specs=[
        _split_spec(),
        _split_spec(),
        _row_spec(1),
        _full_spec((1, OUT_DIM)),
        _full_spec((OUT_DIM, IN_DIM)),
        _full_spec((1, IN_DIM)),
    ],
    out_specs=[_row_spec(OUT_DIM), _row_spec(IN_DIM)],
    out_shape=[
        jax.ShapeDtypeStruct((N_PAD, OUT_DIM), jnp.float32),
        jax.ShapeDtypeStruct((N_PAD, IN_DIM), jnp.float32),
    ],
)


def kernel(x, edge_index, W1, b1, W2, b2, Wd, bd):
    src = edge_index[0]
    dst = edge_index[1]
    padd = E_PAD_D - E
    dst3d = jnp.concatenate([dst, jnp.full((padd,), N, jnp.int32)]).reshape(
        NW, NBD, LANES
    )
    pada = E_PAD_A - E
    src3 = jnp.concatenate([src, jnp.full((pada,), N + 1, jnp.int32)]).reshape(
        NS, NB, LANES
    )
    dst3 = jnp.concatenate([dst, jnp.full((pada,), N, jnp.int32)]).reshape(
        NS, NB, LANES
    )
    x_pad = jnp.concatenate([x, jnp.zeros((N_PAD - N, IN_DIM), x.dtype)])

    cnt = _deg_kernel(dst3d)
    dinv, u1 = _tc0(cnt.T, x_pad)
    s1 = _agg_kernel(u1, src3, dst3)
    (u2,) = _tc1(s1, u1, dinv, W1, b1.reshape(1, HIDDEN), W2)
    s2 = _agg_kernel(u2, src3, dst3)
    emb_pad, recon_pad = _tc2(
        s2, u2, dinv, b2.reshape(1, OUT_DIM), Wd, bd.reshape(1, IN_DIM)
    )
    return emb_pad[:N], recon_pad[:N]


# speedup vs baseline: 1.5049x; 1.4587x over previous
import functools

import jax
import jax.numpy as jnp
from jax import lax
from jax.experimental import pallas as pl
from jax.experimental.pallas import tpu as pltpu
from jax.experimental.pallas import tpu_sc as plsc

N = 10000
E = 320000
IN_DIM = 128
HIDDEN = 256
OUT_DIM = 128
HALF = IN_DIM // 2

NC = 2
NS = 16
NW = NC * NS
LANES = 128

NBD = 80
E_PAD_D = NW * NBD * LANES

NB = 158
E_PAD_A = NS * NB * LANES

N_PAD = 10240
ROWS_PER_TILE = N_PAD // NS
WB_CHUNK = 128
BLK = 1024

_mesh = plsc.VectorSubcoreMesh(
    core_axis_name="c", subcore_axis_name="s", num_cores=NC, num_subcores=NS
)


@functools.partial(
    pl.kernel,
    out_type=jax.ShapeDtypeStruct((NC, N_PAD), jnp.float32),
    mesh=_mesh,
    scratch_types=[
        pltpu.VMEM((NBD, LANES), jnp.int32),
        pltpu.VMEM((LANES,), jnp.float32),
        pltpu.VMEM((ROWS_PER_TILE,), jnp.float32),
        pltpu.VMEM_SHARED((N_PAD,), jnp.float32),
    ],
)
def _deg_kernel(dst_hbm, out_hbm, dst_v, ones_v, cbuf, cnt_acc):
    c = lax.axis_index("c")
    s = lax.axis_index("s")
    g = c * NS + s

    pltpu.sync_copy(dst_hbm.at[g], dst_v)

    def fill_ones(i, carry):
        ones_v[pl.ds(i * 16, 16)] = jnp.ones((16,), jnp.float32)
        return carry

    lax.fori_loop(0, LANES // 16, fill_ones, 0)

    def zero_cbuf(i, carry):
        cbuf[pl.ds(i * 16, 16)] = jnp.zeros((16,), jnp.float32)
        return carry

    lax.fori_loop(0, ROWS_PER_TILE // 16, zero_cbuf, 0)

    pltpu.sync_copy(cbuf, cnt_acc.at[pl.ds(s * ROWS_PER_TILE, ROWS_PER_TILE)])
    plsc.subcore_barrier()

    def count_batch(j, carry):
        pltpu.sync_copy(ones_v, cnt_acc.at[dst_v.at[j]], add=True)
        return carry

    lax.fori_loop(0, NBD, count_batch, 0)
    plsc.subcore_barrier()

    pltpu.sync_copy(cnt_acc.at[pl.ds(s * ROWS_PER_TILE, ROWS_PER_TILE)], cbuf)
    pltpu.sync_copy(cbuf, out_hbm.at[c, pl.ds(s * ROWS_PER_TILE, ROWS_PER_TILE)])


@functools.partial(
    pl.kernel,
    out_type=jax.ShapeDtypeStruct((NC, N_PAD, HALF), jnp.float32),
    mesh=_mesh,
    scratch_types=[
        pltpu.VMEM((NB, LANES), jnp.int32),
        pltpu.VMEM((NB, LANES), jnp.int32),
        pltpu.VMEM((LANES, HALF), jnp.float32),
        pltpu.VMEM((LANES, HALF), jnp.float32),
        pltpu.VMEM_SHARED((N_PAD, HALF), jnp.float32),
        pltpu.SemaphoreType.DMA,
        pltpu.SemaphoreType.DMA,
    ],
    compiler_params=pltpu.CompilerParams(use_tc_tiling_on_sc=False),
)
def _agg_kernel(u_hbm, src_hbm, dst_hbm, out_hbm, src_v, dst_v, bufa, bufb, acc, sema, semb):
    c = lax.axis_index("c")
    s = lax.axis_index("s")

    pltpu.sync_copy(src_hbm.at[s], src_v)
    pltpu.sync_copy(dst_hbm.at[s], dst_v)

    def zero_buf(i, carry):
        bufa[i // (HALF // 16), pl.ds((i % (HALF // 16)) * 16, 16)] = jnp.zeros(
            (16,), jnp.float32
        )
        return carry

    lax.fori_loop(0, LANES * HALF // 16, zero_buf, 0)
    base = s * ROWS_PER_TILE

    def zero_acc(k, carry):
        pltpu.sync_copy(bufa, acc.at[pl.ds(base + k * WB_CHUNK, WB_CHUNK)])
        return carry

    lax.fori_loop(0, ROWS_PER_TILE // WB_CHUNK, zero_acc, 0)
    plsc.subcore_barrier()

    uc = u_hbm.at[c]
    pltpu.async_copy(uc.at[src_v.at[0]], bufa, sema)

    def body(jj, carry):
        j0 = 2 * jj
        j1 = j0 + 1
        pltpu.async_copy(uc.at[src_v.at[j1]], bufb, semb)
        pltpu.make_async_copy(uc.at[src_v.at[j0]], bufa, sema).wait()
        pltpu.sync_copy(bufa, acc.at[dst_v.at[j0]], add=True)

        @pl.when(jj + 1 < NB // 2)
        def _():
            pltpu.async_copy(uc.at[src_v.at[j0 + 2]], bufa, sema)

        pltpu.make_async_copy(uc.at[src_v.at[j1]], bufb, semb).wait()
        pltpu.sync_copy(bufb, acc.at[dst_v.at[j1]], add=True)
        return carry

    lax.fori_loop(0, NB // 2, body, 0)
    plsc.subcore_barrier()

    def writeback(k, carry):
        r0 = base + k * WB_CHUNK
        pltpu.sync_copy(acc.at[pl.ds(r0, WB_CHUNK)], bufa)
        pltpu.sync_copy(bufa, out_hbm.at[c, pl.ds(r0, WB_CHUNK)])
        return carry

    lax.fori_loop(0, ROWS_PER_TILE // WB_CHUNK, writeback, 0)


def _tc0_body(cnt_ref, x_ref, dinv_ref, u_ref):
    cnt = cnt_ref[...]
    deg = cnt[:, 0:1] + cnt[:, 1:2] + 1.0
    dinv = lax.rsqrt(deg)
    dinv_ref[...] = dinv
    u = x_ref[...] * dinv
    u_ref[0] = u[:, :HALF]
    u_ref[1] = u[:, HALF:]


def _tc1_body(s1_ref, u1_ref, dinv_ref, w1_ref, b1_ref, w2_ref, u2_ref):
    dinv = dinv_ref[...]
    su = jnp.concatenate([s1_ref[0] + u1_ref[0], s1_ref[1] + u1_ref[1]], axis=-1)
    p = su * dinv
    h = jnp.dot(p, w1_ref[...], preferred_element_type=jnp.float32) + b1_ref[...]
    h = jnp.maximum(h, 0.0)
    q = jnp.dot(h, w2_ref[...], preferred_element_type=jnp.float32)
    u2 = q * dinv
    u2_ref[0] = u2[:, :HALF]
    u2_ref[1] = u2[:, HALF:]


def _tc2_body(s2_ref, u2_ref, dinv_ref, b2_ref, wd_ref, bd_ref, emb_ref, recon_ref):
    su = jnp.concatenate([s2_ref[0] + u2_ref[0], s2_ref[1] + u2_ref[1]], axis=-1)
    emb = su * dinv_ref[...] + b2_ref[...]
    emb_ref[...] = emb
    recon_ref[...] = (
        jnp.dot(emb, wd_ref[...], preferred_element_type=jnp.float32) + bd_ref[...]
    )


def _row_spec(cols):
    return pl.BlockSpec((BLK, cols), lambda i: (i, 0))


def _split_spec():
    return pl.BlockSpec((NC, BLK, HALF), lambda i: (0, i, 0))


def _full_spec(shape):
    nd = len(shape)
    return pl.BlockSpec(shape, lambda i: (0,) * nd)


_split_shape = jax.ShapeDtypeStruct((NC, N_PAD, HALF), jnp.float32)

_tc0 = pl.pallas_call(
    _tc0_body,
    grid=(N_PAD // BLK,),
    in_specs=[_row_spec(NC), _row_spec(IN_DIM)],
    out_specs=[_row_spec(1), _split_spec()],
    out_shape=[jax.ShapeDtypeStruct((N_PAD, 1), jnp.float32), _split_shape],
)

_tc1 = pl.pallas_call(
    _tc1_body,
    grid=(N_PAD // BLK,),
    in_specs=[
        _split_spec(),
        _split_spec(),
        _row_spec(1),
        _full_spec((IN_DIM, HIDDEN)),
        _full_spec((1, HIDDEN)),
        _full_spec((HIDDEN, OUT_DIM)),
    ],
    out_specs=[_split_spec()],
    out_shape=[_split_shape],
)

_tc2 = pl.pallas_call(
    _tc2_body,
    grid=(N_PAD // BLK,),
    in_specs=[
        _split_spec(),
        _split_spec(),
        _row_spec(1),
        _full_spec((1, OUT_DIM)),
        _full_spec((OUT_DIM, IN_DIM)),
        _full_spec((1, IN_DIM)),
    ],
    out_specs=[_row_spec(OUT_DIM), _row_spec(IN_DIM)],
    out_shape=[
        jax.ShapeDtypeStruct((N_PAD, OUT_DIM), jnp.float32),
        jax.ShapeDtypeStruct((N_PAD, IN_DIM), jnp.float32),
    ],
)


def kernel(x, edge_index, W1, b1, W2, b2, Wd, bd):
    src = edge_index[0]
    dst = edge_index[1]
    padd = E_PAD_D - E
    dst3d = jnp.concatenate([dst, jnp.full((padd,), N, jnp.int32)]).reshape(
        NW, NBD, LANES
    )
    pada = E_PAD_A - E
    src3 = jnp.concatenate([src, jnp.full((pada,), N + 1, jnp.int32)]).reshape(
        NS, NB, LANES
    )
    dst3 = jnp.concatenate([dst, jnp.full((pada,), N, jnp.int32)]).reshape(
        NS, NB, LANES
    )
    x_pad = jnp.concatenate([x, jnp.zeros((N_PAD - N, IN_DIM), x.dtype)])

    cnt = _deg_kernel(dst3d)
    dinv, u1 = _tc0(cnt.T, x_pad)
    s1 = _agg_kernel(u1, src3, dst3)
    (u2,) = _tc1(s1, u1, dinv, W1, b1.reshape(1, HIDDEN), W2)
    s2 = _agg_kernel(u2, src3, dst3)
    emb_pad, recon_pad = _tc2(
        s2, u2, dinv, b2.reshape(1, OUT_DIM), Wd, bd.reshape(1, IN_DIM)
    )
    return emb_pad[:N], recon_pad[:N]
